# baseline (speedup 1.0000x reference)
import jax
import jax.numpy as jnp
from jax.experimental import pallas as pl
from jax.experimental.pallas import tpu as pltpu
from jax.experimental.pallas import tpu_sc as plsc

_R = 8


def kernel(inputs, pos_table):
    B, S, D = inputs.shape
    N = B * S
    n_tab = S // _R
    x2d = inputs.reshape(N, D)
    mesh = plsc.VectorSubcoreMesh(core_axis_name="c", subcore_axis_name="s")

    @pl.kernel(out_type=jax.ShapeDtypeStruct((N, D), inputs.dtype), mesh=mesh)
    def sc_add(x_hbm, t_hbm, o_hbm):
        def body(x_vmem, t_vmem, o_vmem):
            @pl.loop(0, _R)
            def _(r):
                @pl.loop(0, D, step=16)
                def _(c):
                    o_vmem.at[r, pl.ds(c, 16)][...] = (
                        x_vmem.at[r, pl.ds(c, 16)][...]
                        + t_vmem.at[r, pl.ds(c, 16)][...]
                    )

        pltpu.emit_pipeline(
            body,
            grid=(N // _R,),
            in_specs=[
                pl.BlockSpec((_R, D), lambda i: (i, 0)),
                pl.BlockSpec((_R, D), lambda i: (jax.lax.rem(i, n_tab), 0)),
            ],
            out_specs=[pl.BlockSpec((_R, D), lambda i: (i, 0))],
            core_axis_name=("c", "s"),
            dimension_semantics=(pltpu.PARALLEL,),
        )(x_hbm, t_hbm, o_hbm)

    return sc_add(x2d, pos_table).reshape(B, S, D)
